# baseline (speedup 1.0000x reference)
import jax
import jax.numpy as jnp
from jax.experimental import pallas as pl

T = 32768
D = 1024
E = 64
K = 8
BLK = 2048
INT_MIN = -2147483648
MASK7F = 0x7FFFFFFF


def _router_block(x_ref, wt_ref, b_ref, idx_ref, w_ref):
    x = x_ref[...]
    wt = wt_ref[...]
    logits = jax.lax.dot_general(
        x, wt, (((1,), (0,)), ((), ())), preferred_element_type=jnp.float32
    ) + b_ref[...]
    m0 = jnp.max(logits, axis=1, keepdims=True)
    p = jnp.exp(logits - m0)
    denom = jnp.sum(p, axis=1, keepdims=True)
    lane = jax.lax.broadcasted_iota(jnp.int32, (BLK, E), 1)
    li = jax.lax.bitcast_convert_type(logits, jnp.int32)
    key = jnp.where(li >= 0, li, li ^ jnp.int32(MASK7F))
    key = (key & jnp.int32(-64)) | (jnp.int32(E - 1) - lane)
    best = []
    for _ in range(K):
        mk = jnp.max(key, axis=1, keepdims=True)
        best.append(mk)
        key = jnp.where(key == mk, jnp.int32(INT_MIN), key)
    idx_cols = [jnp.int32(E - 1) - (mk & jnp.int32(E - 1)) for mk in best]
    val_cols = [
        jnp.max(jnp.where(lane == ic, p, -1.0), axis=1, keepdims=True)
        for ic in idx_cols
    ]
    inv_denom = 1.0 / denom
    s_cols = [v * inv_denom for v in val_cols]
    for k in range(K - 1):
        swap = s_cols[k + 1] > s_cols[k]
        s_cols[k], s_cols[k + 1] = (
            jnp.where(swap, s_cols[k + 1], s_cols[k]),
            jnp.where(swap, s_cols[k], s_cols[k + 1]),
        )
        idx_cols[k], idx_cols[k + 1] = (
            jnp.where(swap, idx_cols[k + 1], idx_cols[k]),
            jnp.where(swap, idx_cols[k], idx_cols[k + 1]),
        )
    idx_ref[...] = jnp.concatenate(idx_cols, axis=1)
    w_ref[...] = jnp.concatenate(s_cols, axis=1)


@jax.jit
def kernel(x, W, b):
    wt = W.T
    b2 = b.reshape(1, E)
    grid = (T // BLK,)
    return pl.pallas_call(
        _router_block,
        grid=grid,
        in_specs=[
            pl.BlockSpec((BLK, D), lambda i: (i, 0)),
            pl.BlockSpec((D, E), lambda i: (0, 0)),
            pl.BlockSpec((1, E), lambda i: (0, 0)),
        ],
        out_specs=[
            pl.BlockSpec((BLK, K), lambda i: (i, 0)),
            pl.BlockSpec((BLK, K), lambda i: (i, 0)),
        ],
        out_shape=[
            jax.ShapeDtypeStruct((T, K), jnp.int32),
            jax.ShapeDtypeStruct((T, K), jnp.float32),
        ],
    )(x, wt, b2)
